# baseline (speedup 1.0000x reference)
import functools

import jax
import jax.numpy as jnp
from jax import lax
from jax.experimental import pallas as pl
from jax.experimental.pallas import tpu as pltpu
from jax.experimental.pallas import tpu_sc as plsc

DIM_ = 2048
NE_ = 64
NG_ = 8
GS_ = NE_ // NG_
TKG_ = 4
TK_ = 8
T_ = 8192

NC_ = 2
NS_ = 16
NW_ = NC_ * NS_
TPW_ = T_ // NW_
L_ = 16
CHUNKS_ = TPW_ // L_

BT_ = 1024


def _scores_body(w_ref, x_ref, o_ref):
    s = lax.dot_general(
        w_ref[...], x_ref[...], (((1,), (1,)), ((), ())),
        preferred_element_type=jnp.float32,
        precision=lax.Precision.DEFAULT,
    )
    m = jnp.max(s, axis=0, keepdims=True)
    p = jnp.exp(s - m)
    p = p / jnp.sum(p, axis=0, keepdims=True)
    gmr = [jnp.max(p[g * GS_:(g + 1) * GS_], axis=0, keepdims=True)
           for g in range(NG_)]
    ninf_row = jnp.full((1, BT_), -jnp.inf, jnp.float32)
    keepr = [ninf_row] * NG_
    for _ in range(TKG_):
        mm = functools.reduce(jnp.maximum, gmr)
        found = jnp.zeros((1, BT_), jnp.int32)
        for g in range(NG_):
            eq = jnp.logical_and(gmr[g] == mm, found == 0)
            keepr[g] = jnp.where(eq, 0.0, keepr[g])
            found = jnp.where(eq, 1, found)
            gmr[g] = jnp.where(eq, ninf_row, gmr[g])
    pb = lax.bitcast_convert_type(p, jnp.int32)
    eidx = lax.broadcasted_iota(jnp.int32, (NE_, BT_), 0)
    kbits = jnp.bitwise_or(jnp.bitwise_and(pb, ~63), 63 - eidx)
    keepx = jnp.concatenate(
        [jnp.broadcast_to(keepr[g], (GS_, BT_)) for g in range(NG_)], axis=0)
    o_ref[...] = lax.bitcast_convert_type(kbits, jnp.float32) + keepx


def _scores_tc(x, weight):
    ts = x.shape[0]
    return pl.pallas_call(
        _scores_body,
        grid=(ts // BT_,),
        in_specs=[
            pl.BlockSpec((NE_, DIM_), lambda i: (0, 0)),
            pl.BlockSpec((BT_, DIM_), lambda i: (i, 0)),
        ],
        out_specs=pl.BlockSpec((NE_, BT_), lambda i: (0, i)),
        out_shape=jax.ShapeDtypeStruct((NE_, ts), jnp.float32),
    )(weight, x)


def _route_body(tpw, sT_hbm, wT_hbm, iT_hbm, s_v, w_v, i_v, sem):
    wid = lax.axis_index("s") * NC_ + lax.axis_index("c")
    base = wid * tpw
    half = tpw // 2
    pltpu.sync_copy(sT_hbm.at[:, pl.ds(base, half)], s_v.at[:, pl.ds(0, half)])
    cp2 = pltpu.async_copy(sT_hbm.at[:, pl.ds(base + half, half)],
                           s_v.at[:, pl.ds(half, half)], sem)
    ninf = jnp.full((L_,), -jnp.inf, jnp.float32)
    c63 = jnp.full((L_,), 63, jnp.int32)
    cm64 = jnp.full((L_,), ~63, jnp.int32)

    def chunk(c, carry):
        t0 = pl.multiple_of(c * 2 * L_, L_)
        t1 = pl.multiple_of(t0 + L_, L_)

        def octet(ee, bs):
            b0 = list(bs[:TK_])
            b1 = list(bs[TK_:])
            for j in range(GS_):
                key0 = s_v[ee * GS_ + j, pl.ds(t0, L_)]
                key1 = s_v[ee * GS_ + j, pl.ds(t1, L_)]
                for k in range(TK_ - 1, 0, -1):
                    b0[k] = jnp.maximum(b0[k], jnp.minimum(b0[k - 1], key0))
                    b1[k] = jnp.maximum(b1[k], jnp.minimum(b1[k - 1], key1))
                b0[0] = jnp.maximum(b0[0], key0)
                b1[0] = jnp.maximum(b1[0], key1)
            return tuple(b0 + b1)

        b = lax.fori_loop(0, NE_ // GS_, octet, (ninf,) * (2 * TK_))
        for r in range(TK_):
            for (bt, tt) in ((b[r], t0), (b[TK_ + r], t1)):
                bb = lax.bitcast_convert_type(bt, jnp.int32)
                w_v[r, pl.ds(tt, L_)] = lax.bitcast_convert_type(
                    jnp.bitwise_and(bb, cm64), jnp.float32)
                i_v[r, pl.ds(tt, L_)] = c63 - jnp.bitwise_and(bb, c63)
        return carry

    lax.fori_loop(0, half // (2 * L_), chunk, 0)
    cp2.wait()
    lax.fori_loop(half // (2 * L_), tpw // (2 * L_), chunk, 0)
    pltpu.sync_copy(w_v, wT_hbm.at[:, pl.ds(base, tpw)])
    pltpu.sync_copy(i_v, iT_hbm.at[:, pl.ds(base, tpw)])


def _route_sc(scores_t):
    ts = scores_t.shape[1]
    tpw = ts // NW_
    mesh = plsc.VectorSubcoreMesh(core_axis_name="c", subcore_axis_name="s")
    f = functools.partial(
        pl.kernel,
        mesh=mesh,
        out_type=[
            jax.ShapeDtypeStruct((TK_, ts), jnp.float32),
            jax.ShapeDtypeStruct((TK_, ts), jnp.int32),
        ],
        scratch_types=[
            pltpu.VMEM((NE_, tpw), jnp.float32),
            pltpu.VMEM((TK_, tpw), jnp.float32),
            pltpu.VMEM((TK_, tpw), jnp.int32),
            pltpu.SemaphoreType.DMA,
        ],
    )(functools.partial(_route_body, tpw))
    return f(scores_t)


NSLICE_ = 1


def kernel(x, weight):
    outs = []
    ts = T_ // NSLICE_
    for s in range(NSLICE_):
        st = _scores_tc(lax.slice_in_dim(x, s * ts, (s + 1) * ts, axis=0),
                        weight)
        outs.append(_route_sc(st))
    if NSLICE_ == 1:
        w_t, i_t = outs[0]
    else:
        w_t = jnp.concatenate([o[0] for o in outs], axis=1)
        i_t = jnp.concatenate([o[1] for o in outs], axis=1)
    return (w_t.T, i_t.T)


# speedup vs baseline: 1.0127x; 1.0127x over previous
import functools

import jax
import jax.numpy as jnp
from jax import lax
from jax.experimental import pallas as pl
from jax.experimental.pallas import tpu as pltpu
from jax.experimental.pallas import tpu_sc as plsc

DIM_ = 2048
NE_ = 64
NG_ = 8
GS_ = NE_ // NG_
TKG_ = 4
TK_ = 8
T_ = 8192

NC_ = 2
NS_ = 16
NW_ = NC_ * NS_
TPW_ = T_ // NW_
L_ = 16
CHUNKS_ = TPW_ // L_

BT_ = 1024


def _scores_body(w_ref, x_ref, o_ref):
    s = lax.dot_general(
        w_ref[...], x_ref[...], (((1,), (1,)), ((), ())),
        preferred_element_type=jnp.float32,
        precision=lax.Precision.DEFAULT,
    )
    m = jnp.max(s, axis=0, keepdims=True)
    p = jnp.exp(s - m)
    p = p / jnp.sum(p, axis=0, keepdims=True)
    gmr = [jnp.max(p[g * GS_:(g + 1) * GS_], axis=0, keepdims=True)
           for g in range(NG_)]
    ninf_row = jnp.full((1, BT_), -jnp.inf, jnp.float32)
    keepr = [ninf_row] * NG_
    for _ in range(TKG_):
        mm = functools.reduce(jnp.maximum, gmr)
        found = jnp.zeros((1, BT_), jnp.int32)
        for g in range(NG_):
            eq = jnp.logical_and(gmr[g] == mm, found == 0)
            keepr[g] = jnp.where(eq, 0.0, keepr[g])
            found = jnp.where(eq, 1, found)
            gmr[g] = jnp.where(eq, ninf_row, gmr[g])
    pb = lax.bitcast_convert_type(p, jnp.int32)
    eidx = lax.broadcasted_iota(jnp.int32, (NE_, BT_), 0)
    kbits = jnp.bitwise_or(jnp.bitwise_and(pb, ~63), 63 - eidx)
    keepx = jnp.concatenate(
        [jnp.broadcast_to(keepr[g], (GS_, BT_)) for g in range(NG_)], axis=0)
    o_ref[...] = lax.bitcast_convert_type(kbits, jnp.float32) + keepx


def _scores_tc(x, weight):
    ts = x.shape[0]
    return pl.pallas_call(
        _scores_body,
        grid=(ts // BT_,),
        in_specs=[
            pl.BlockSpec((NE_, DIM_), lambda i: (0, 0)),
            pl.BlockSpec((BT_, DIM_), lambda i: (i, 0)),
        ],
        out_specs=pl.BlockSpec((NE_, BT_), lambda i: (0, i)),
        out_shape=jax.ShapeDtypeStruct((NE_, ts), jnp.float32),
    )(weight, x)


def _route_body(tpw, sT_hbm, wT_hbm, iT_hbm, s_v, w_v, i_v, sem):
    wid = lax.axis_index("s") * NC_ + lax.axis_index("c")
    base = wid * tpw
    half = tpw // 2
    pltpu.sync_copy(sT_hbm.at[:, pl.ds(base, half)], s_v.at[:, pl.ds(0, half)])
    cp2 = pltpu.async_copy(sT_hbm.at[:, pl.ds(base + half, half)],
                           s_v.at[:, pl.ds(half, half)], sem)
    ninf = jnp.full((L_,), -jnp.inf, jnp.float32)
    c63 = jnp.full((L_,), 63, jnp.int32)
    cm64 = jnp.full((L_,), ~63, jnp.int32)

    def chunk(c, carry):
        t0 = pl.multiple_of(c * L_, L_)
        def octet(ee, bs):
            bs = list(bs)
            for j in range(GS_):
                key = s_v[ee * GS_ + j, pl.ds(t0, L_)]
                for k in range(TK_ - 1, 0, -1):
                    bs[k] = jnp.maximum(bs[k], jnp.minimum(bs[k - 1], key))
                bs[0] = jnp.maximum(bs[0], key)
            return tuple(bs)

        b = lax.fori_loop(0, NE_ // GS_, octet, (ninf,) * TK_)
        for r in range(TK_):
            bb = lax.bitcast_convert_type(b[r], jnp.int32)
            w_v[r, pl.ds(t0, L_)] = lax.bitcast_convert_type(
                jnp.bitwise_and(bb, cm64), jnp.float32)
            i_v[r, pl.ds(t0, L_)] = c63 - jnp.bitwise_and(bb, c63)
        return carry

    lax.fori_loop(0, half // L_, chunk, 0)
    cp2.wait()
    lax.fori_loop(half // L_, tpw // L_, chunk, 0)
    pltpu.sync_copy(w_v, wT_hbm.at[:, pl.ds(base, tpw)])
    pltpu.sync_copy(i_v, iT_hbm.at[:, pl.ds(base, tpw)])


def _route_sc(scores_t):
    ts = scores_t.shape[1]
    tpw = ts // NW_
    mesh = plsc.VectorSubcoreMesh(core_axis_name="c", subcore_axis_name="s")
    f = functools.partial(
        pl.kernel,
        mesh=mesh,
        out_type=[
            jax.ShapeDtypeStruct((TK_, ts), jnp.float32),
            jax.ShapeDtypeStruct((TK_, ts), jnp.int32),
        ],
        scratch_types=[
            pltpu.VMEM((NE_, tpw), jnp.float32),
            pltpu.VMEM((TK_, tpw), jnp.float32),
            pltpu.VMEM((TK_, tpw), jnp.int32),
            pltpu.SemaphoreType.DMA,
        ],
    )(functools.partial(_route_body, tpw))
    return f(scores_t)


NSLICE_ = 1


def kernel(x, weight):
    outs = []
    ts = T_ // NSLICE_
    for s in range(NSLICE_):
        st = _scores_tc(lax.slice_in_dim(x, s * ts, (s + 1) * ts, axis=0),
                        weight)
        outs.append(_route_sc(st))
    if NSLICE_ == 1:
        w_t, i_t = outs[0]
    else:
        w_t = jnp.concatenate([o[0] for o in outs], axis=1)
        i_t = jnp.concatenate([o[1] for o in outs], axis=1)
    return (w_t.T, i_t.T)
